# baseline (speedup 1.0000x reference)
import functools

import jax
import jax.numpy as jnp
from jax import lax
from jax.experimental import pallas as pl
from jax.experimental.pallas import tpu as pltpu
from jax.experimental.pallas import tpu_sc as plsc

N = 262144
D = 32
K = 64
L = 16
NC = 2
NS = 16
NW = NC * NS
PPW = N // NW
CHUNK = 1024
NCHUNK = PPW // CHUNK
GROUPS = CHUNK // L
LOSS_SCALE = 0.1


def _make_sc_call():
    mesh = plsc.VectorSubcoreMesh(core_axis_name="c", subcore_axis_name="s")

    @functools.partial(
        pl.kernel,
        mesh=mesh,
        out_type=jax.ShapeDtypeStruct((NW, L), jnp.float32),
        compiler_params=pltpu.CompilerParams(
            needs_layout_passes=False, use_tc_tiling_on_sc=False),
        scratch_types=[
            pltpu.VMEM((CHUNK * D,), jnp.float32),
            pltpu.VMEM((K, D, L), jnp.float32),
            pltpu.VMEM((K, L), jnp.float32),
            pltpu.VMEM((L,), jnp.float32),
            pltpu.VMEM((L,), jnp.float32),
        ],
    )
    def sc_kernel(e_hbm, cb_hbm, a_hbm, out_hbm, ebuf, cbuf, tbuf, abuf, lbuf):
        wid = lax.axis_index("s") * NC + lax.axis_index("c")
        base = wid * PPW

        pltpu.sync_copy(cb_hbm, cbuf)
        pltpu.sync_copy(a_hbm, abuf)
        neg_alpha = -abuf[...]
        lane_off = lax.iota(jnp.int32, L) * D

        acc = jnp.zeros((L,), jnp.float32)
        for c in range(NCHUNK):
            pltpu.sync_copy(
                e_hbm.at[pl.ds((base + c * CHUNK) * D, CHUNK * D)], ebuf)

            def g_body(g, acc):
                gbase = lane_off + g * (L * D)
                evecs = [plsc.load_gather(ebuf, [gbase + d]) for d in range(D)]

                def k_body(k, m):
                    dist = jnp.zeros((L,), jnp.float32)
                    for d in range(D):
                        diff = evecs[d] - cbuf[k, d, :]
                        dist = dist + diff * diff
                    tbuf[k, :] = dist
                    return jnp.minimum(m, dist)

                m = lax.fori_loop(0, K, k_body,
                                  jnp.full((L,), jnp.inf, jnp.float32))

                def s_body(k, carry):
                    s, num = carry
                    dk = tbuf[k, :]
                    w = jnp.exp(neg_alpha * (dk - m))
                    return s + w, num + dk * w

                s, num = lax.fori_loop(
                    0, K, s_body,
                    (jnp.zeros((L,), jnp.float32), jnp.zeros((L,), jnp.float32)))
                return acc + num / s

            acc = lax.fori_loop(0, GROUPS, g_body, acc)

        lbuf[...] = acc
        pltpu.sync_copy(lbuf, out_hbm.at[wid])

    return sc_kernel


_SC_KERNEL = _make_sc_call()


def kernel(embeddings, centroids, alpha):
    cb = jnp.broadcast_to(
        centroids.astype(jnp.float32)[:, :, None], (K, D, L))
    avec = jnp.full((L,), alpha, dtype=jnp.float32)
    partials = _SC_KERNEL(embeddings.reshape(N * D), cb, avec)
    return jnp.sum(partials) * jnp.float32(LOSS_SCALE / N)
